# Initial kernel scaffold; baseline (speedup 1.0000x reference)
import jax
import jax.numpy as jnp
from jax.experimental import pallas as pl


def kernel(X, ref_a, ref_b, W_hidden, W_out, b_hidden, b_out):
    raise NotImplementedError("write your pallas kernel here")


# speedup vs baseline: 7.7518x; 7.7518x over previous
import functools

import jax
import jax.numpy as jnp
from jax import lax
from jax.experimental import pallas as pl
from jax.experimental.pallas import tpu as pltpu
from jax.experimental.pallas import tpu_sc as plsc

_NUM_CORES = 2
_NUM_SUBCORES = 16
_NW = _NUM_CORES * _NUM_SUBCORES
_CHUNK = 80


@functools.partial(jax.jit, static_argnums=(3, 4, 5))
def _scatter_partials(X, a3, b3, N_pad, D, n_chunks):
    rows_per_tile = N_pad // _NUM_SUBCORES
    n_zcopies = rows_per_tile // _CHUNK
    mesh = plsc.VectorSubcoreMesh(core_axis_name="c", subcore_axis_name="s")

    @functools.partial(
        pl.kernel,
        out_type=jax.ShapeDtypeStruct((_NUM_CORES, N_pad, D), jnp.float32),
        mesh=mesh,
        scratch_types=[
            pltpu.VMEM((n_chunks, _CHUNK), jnp.int32),
            pltpu.VMEM((n_chunks, _CHUNK), jnp.int32),
            pltpu.VMEM((_CHUNK, D), jnp.float32),
            pltpu.VMEM_SHARED((N_pad, D), jnp.float32),
            pltpu.SemaphoreType.DMA,
        ],
    )
    def sc_kernel(x_hbm, a_hbm, b_hbm, out_hbm, idx_a, idx_b, rows, acc, sem):
        c = lax.axis_index("c")
        s = lax.axis_index("s")
        wid = s * _NUM_CORES + c

        zero = jnp.zeros((16,), jnp.float32)

        def zero_row(r, _):
            for j in range(D // 16):
                rows[r, pl.ds(j * 16, 16)] = zero
            return 0

        lax.fori_loop(0, _CHUNK, zero_row, 0)
        base = s * rows_per_tile
        for k in range(n_zcopies):
            pltpu.sync_copy(rows, acc.at[pl.ds(base + k * _CHUNK, _CHUNK)])

        pltpu.sync_copy(a_hbm.at[wid], idx_a)
        pltpu.sync_copy(b_hbm.at[wid], idx_b)

        plsc.subcore_barrier()

        def chunk_body(i, _):
            pltpu.async_copy(x_hbm.at[idx_a.at[i]], rows, sem).wait()
            pltpu.sync_copy(rows, acc.at[idx_b.at[i]], add=True)
            return 0

        lax.fori_loop(0, n_chunks, chunk_body, 0)

        plsc.subcore_barrier()

        pltpu.sync_copy(
            acc.at[pl.ds(base, rows_per_tile)],
            out_hbm.at[c, pl.ds(base, rows_per_tile)],
        )

    return sc_kernel(X, a3, b3)


def _mlp(p0, p1, W_hidden, W_out, b_hidden, b_out):
    N, D = p0.shape
    U = W_out.shape[1]
    blk = 2000

    def tc_kernel(p0_ref, p1_ref, wh_ref, wo_ref, bh_ref, bo_ref, o_ref):
        agg = p0_ref[...] + p1_ref[...]
        hid = jnp.dot(agg, wh_ref[...], preferred_element_type=jnp.float32)
        hid = hid + bh_ref[...]
        out = jnp.dot(hid, wo_ref[...], preferred_element_type=jnp.float32)
        o_ref[...] = out + bo_ref[...]

    return pl.pallas_call(
        tc_kernel,
        grid=(N // blk,),
        in_specs=[
            pl.BlockSpec((blk, D), lambda i: (i, 0)),
            pl.BlockSpec((blk, D), lambda i: (i, 0)),
            pl.BlockSpec((D, W_hidden.shape[1]), lambda i: (0, 0)),
            pl.BlockSpec((W_out.shape[0], U), lambda i: (0, 0)),
            pl.BlockSpec((1, W_hidden.shape[1]), lambda i: (0, 0)),
            pl.BlockSpec((1, U), lambda i: (0, 0)),
        ],
        out_specs=pl.BlockSpec((blk, U), lambda i: (i, 0)),
        out_shape=jax.ShapeDtypeStruct((N, U), jnp.float32),
    )(p0, p1, W_hidden, W_out, b_hidden.reshape(1, -1), b_out.reshape(1, -1))


def kernel(X, ref_a, ref_b, W_hidden, W_out, b_hidden, b_out):
    N, D = X.shape
    E = ref_a.shape[0]
    e_per_w = E // _NW
    n_chunks = e_per_w // _CHUNK
    n_pad = -(-N // (_NUM_SUBCORES * 128)) * (_NUM_SUBCORES * 128)
    a3 = ref_a.astype(jnp.int32).reshape(_NW, n_chunks, _CHUNK)
    b3 = ref_b.astype(jnp.int32).reshape(_NW, n_chunks, _CHUNK)
    partials = _scatter_partials(X, a3, b3, n_pad, D, n_chunks)
    p0 = partials[0, :N]
    p1 = partials[1, :N]
    return _mlp(p0, p1, W_hidden, W_out, b_hidden, b_out)
